# baseline (speedup 1.0000x reference)
import functools
import jax
import jax.numpy as jnp
from jax import lax
from jax.experimental import pallas as pl
from jax.experimental.pallas import tpu as pltpu, tpu_sc as plsc

B, D, T = 4, 1024, 2048
TW = 512
DW = 128
DC = 4
NCH = DW // DC
NQ = 3
NO = 2


def _sc_body(q_hbm, pos_hbm, out_hbm, stage_v, posT_v, q_v, o_v,
             sem_p, sem_q, sem_o):
    c = lax.axis_index("c")
    s = lax.axis_index("s")
    tix = s % 4
    dix = (s // 4) + c * 4
    t0 = tix * TW
    d0 = dix * DW

    def start_q(buf, i):
        return pltpu.async_copy(
            q_hbm.at[:, pl.ds(d0 + i * DC, DC), pl.ds(t0, TW)],
            q_v.at[buf],
            sem_q,
        )

    def start_o(buf, i):
        return pltpu.async_copy(
            o_v.at[buf],
            out_hbm.at[:, pl.ds(d0 + i * DC, DC), pl.ds(t0, TW)],
            sem_o,
        )

    def transpose_quarter(p):
        @plsc.parallel_loop(0, 128 * (DW // 16), unroll=8)
        def body(k):
            tt = k // (DW // 16)
            dg = k % (DW // 16)
            reg = stage_v[tt, pl.ds(dg * 16, 16)]
            idx_d = lax.iota(jnp.int32, 16) + dg * 16
            idx_t = jnp.full((16,), p * 128 + tt, jnp.int32)
            plsc.store_scatter(posT_v, [idx_d, idx_t], reg)

    def compute(qbuf, obuf, i):
        @plsc.parallel_loop(0, (TW // 16) * DC, unroll=4)
        def body(k):
            tg = k // DC
            d_local = k % DC
            pos_reg = posT_v[i * DC + d_local, pl.ds(tg * 16, 16)]
            for b in range(B):
                o_v[obuf, b, d_local, pl.ds(tg * 16, 16)] = (
                    q_v[qbuf, b, d_local, pl.ds(tg * 16, 16)] + pos_reg
                )

    load_pend = [None] * NQ
    store_pend = [None] * NO
    load_pend[0] = start_q(0, 0)
    load_pend[1] = start_q(1, 1)
    for p in range(TW // 128):
        pltpu.async_copy(
            pos_hbm.at[pl.ds(t0 + p * 128, 128), pl.ds(d0, DW)],
            stage_v,
            sem_p,
        ).wait()
        transpose_quarter(p)
    for i in range(NCH):
        qbuf = i % NQ
        obuf = i % NO
        if i + 2 < NCH:
            load_pend[(i + 2) % NQ] = start_q((i + 2) % NQ, i + 2)
        load_pend[qbuf].wait()
        if store_pend[obuf] is not None:
            store_pend[obuf].wait()
        compute(qbuf, obuf, i)
        store_pend[obuf] = start_o(obuf, i)
    for pend in store_pend:
        if pend is not None:
            pend.wait()


def kernel(q, pos_weight):
    mesh = plsc.VectorSubcoreMesh(core_axis_name="c", subcore_axis_name="s")
    k = functools.partial(
        pl.kernel,
        mesh=mesh,
        out_type=jax.ShapeDtypeStruct((B, D, T), jnp.float32),
        scratch_types=[
            pltpu.VMEM((128, DW), jnp.float32),
            pltpu.VMEM((DW, TW), jnp.float32),
            pltpu.VMEM((NQ, B, DC, TW), jnp.float32),
            pltpu.VMEM((NO, B, DC, TW), jnp.float32),
            pltpu.SemaphoreType.DMA,
            pltpu.SemaphoreType.DMA,
            pltpu.SemaphoreType.DMA,
        ],
        compiler_params=pltpu.CompilerParams(needs_layout_passes=False),
    )(_sc_body)
    return k(q, pos_weight)


# speedup vs baseline: 1.0315x; 1.0315x over previous
import functools
import jax
import jax.numpy as jnp
from jax import lax
from jax.experimental import pallas as pl
from jax.experimental.pallas import tpu as pltpu, tpu_sc as plsc

B, D, T = 4, 1024, 2048
TW = 512
DW = 128
DC = 8
NCH = DW // DC
TH = 256


def _sc_body(q_hbm, pos_hbm, out_hbm, posT_v, sem_p, sem_q, sem_o):
    c = lax.axis_index("c")
    s = lax.axis_index("s")
    tix = s % 4
    dix = (s // 4) + c * 4
    t0 = tix * TW
    d0 = dix * DW

    def phase1(stage_v):
        for p in range(TW // 128):
            pltpu.async_copy(
                pos_hbm.at[pl.ds(t0 + p * 128, 128), pl.ds(d0, DW)],
                stage_v,
                sem_p,
            ).wait()

            @plsc.parallel_loop(0, 128, unroll=2)
            def body(tt):
                for dg in range(DW // 16):
                    reg = stage_v[tt, pl.ds(dg * 16, 16)]
                    idx_d = lax.iota(jnp.int32, 16) + dg * 16
                    idx_t = jnp.full((16,), p * 128 + tt, jnp.int32)
                    plsc.store_scatter(posT_v, [idx_d, idx_t], reg)

    pl.run_scoped(phase1, pltpu.VMEM((128, DW), jnp.float32))

    def phase2(q_v, o_v):
        def start_q(buf, i):
            return pltpu.async_copy(
                q_hbm.at[:, pl.ds(d0 + i * DC, DC), pl.ds(t0, TW)],
                q_v.at[buf],
                sem_q,
            )

        def start_o(j, i, h):
            return pltpu.async_copy(
                o_v.at[j],
                out_hbm.at[:, pl.ds(d0 + i * DC, DC), pl.ds(t0 + h * TH, TH)],
                sem_o,
            )

        def compute_half(qbuf, j, i, h):
            @plsc.parallel_loop(0, TH // 16, unroll=1)
            def body(tg):
                for dl in range(DC):
                    pos_reg = posT_v[i * DC + dl, pl.ds(h * TH + tg * 16, 16)]
                    for b in range(B):
                        o_v[j, b, dl, pl.ds(tg * 16, 16)] = (
                            q_v[qbuf, b, dl, pl.ds(h * TH + tg * 16, 16)]
                            + pos_reg
                        )

        load_pend = [None, None]
        store_pend = [None, None, None]
        load_pend[0] = start_q(0, 0)
        for i in range(NCH):
            qbuf = i % 2
            if i + 1 < NCH:
                load_pend[(i + 1) % 2] = start_q((i + 1) % 2, i + 1)
            load_pend[qbuf].wait()
            for h in range(2):
                j = (2 * i + h) % 3
                if store_pend[j] is not None:
                    store_pend[j].wait()
                compute_half(qbuf, j, i, h)
                store_pend[j] = start_o(j, i, h)
        for pend in store_pend:
            if pend is not None:
                pend.wait()

    pl.run_scoped(
        phase2,
        pltpu.VMEM((2, B, DC, TW), jnp.float32),
        pltpu.VMEM((3, B, DC, TH), jnp.float32),
    )


def kernel(q, pos_weight):
    mesh = plsc.VectorSubcoreMesh(core_axis_name="c", subcore_axis_name="s")
    k = functools.partial(
        pl.kernel,
        mesh=mesh,
        out_type=jax.ShapeDtypeStruct((B, D, T), jnp.float32),
        scratch_types=[
            pltpu.VMEM((DW, TW), jnp.float32),
            pltpu.SemaphoreType.DMA,
            pltpu.SemaphoreType.DMA,
            pltpu.SemaphoreType.DMA,
        ],
        compiler_params=pltpu.CompilerParams(needs_layout_passes=False),
    )(_sc_body)
    return k(q, pos_weight)


# speedup vs baseline: 1.3363x; 1.2954x over previous
import functools
import jax
import jax.numpy as jnp
from jax import lax
from jax.experimental import pallas as pl
from jax.experimental.pallas import tpu as pltpu, tpu_sc as plsc

B, D, T = 4, 1024, 2048
TW = 512
DW = 128
DC = 4
NCH = DW // DC
NQ = 3
NO = 2


def _sc_body(q_hbm, pos_hbm, out_hbm, pos_v, q_v, o_v, sem_p, sem_q, sem_o):
    c = lax.axis_index("c")
    s = lax.axis_index("s")
    tix = s % 4
    dix = (s // 4) + c * 4
    t0 = tix * TW
    d0 = dix * DW

    def start_q(buf, i):
        return pltpu.async_copy(
            q_hbm.at[:, pl.ds(d0 + i * DC, DC), pl.ds(t0, TW)],
            q_v.at[buf],
            sem_q,
        )

    def start_o(buf, i):
        return pltpu.async_copy(
            o_v.at[buf],
            out_hbm.at[:, pl.ds(d0 + i * DC, DC), pl.ds(t0, TW)],
            sem_o,
        )

    def compute(qbuf, obuf, i):
        @plsc.parallel_loop(0, (TW // 16) * DC, unroll=4)
        def body(k):
            tg = k // DC
            d_local = k % DC
            idx_t = lax.iota(jnp.int32, 16) + tg * 16
            idx_d = jnp.full((16,), i * DC + d_local, jnp.int32)
            pos_reg = plsc.load_gather(pos_v, [idx_t, idx_d])
            for b in range(B):
                o_v[obuf, b, d_local, pl.ds(tg * 16, 16)] = (
                    q_v[qbuf, b, d_local, pl.ds(tg * 16, 16)] + pos_reg
                )

    ph = pltpu.async_copy(
        pos_hbm.at[pl.ds(t0, TW), pl.ds(d0, DW)], pos_v, sem_p
    )
    load_pend = [None] * NQ
    store_pend = [None] * NO
    load_pend[0] = start_q(0, 0)
    load_pend[1] = start_q(1, 1)
    ph.wait()
    for i in range(NCH):
        qbuf = i % NQ
        obuf = i % NO
        if i + 2 < NCH:
            load_pend[(i + 2) % NQ] = start_q((i + 2) % NQ, i + 2)
        load_pend[qbuf].wait()
        if store_pend[obuf] is not None:
            store_pend[obuf].wait()
        compute(qbuf, obuf, i)
        store_pend[obuf] = start_o(obuf, i)
    for pend in store_pend:
        if pend is not None:
            pend.wait()


def kernel(q, pos_weight):
    mesh = plsc.VectorSubcoreMesh(core_axis_name="c", subcore_axis_name="s")
    k = functools.partial(
        pl.kernel,
        mesh=mesh,
        out_type=jax.ShapeDtypeStruct((B, D, T), jnp.float32),
        scratch_types=[
            pltpu.VMEM((TW, DW), jnp.float32),
            pltpu.VMEM((NQ, B, DC, TW), jnp.float32),
            pltpu.VMEM((NO, B, DC, TW), jnp.float32),
            pltpu.SemaphoreType.DMA,
            pltpu.SemaphoreType.DMA,
            pltpu.SemaphoreType.DMA,
        ],
        compiler_params=pltpu.CompilerParams(needs_layout_passes=False),
    )(_sc_body)
    return k(q, pos_weight)
